# baseline (speedup 1.0000x reference)
import jax
import jax.numpy as jnp
from jax import lax
from jax.experimental import pallas as pl
from jax.experimental.pallas import tpu as pltpu
from jax.experimental.pallas import tpu_sc as plsc

N = 10000
F = 128
E = 320000
NC = 2
NS = 16
NW = NC * NS
PW = E // NW
C = 50
NCHUNK = PW // C
RB = 624
SC = 20
NSUP = NCHUNK // SC
ZR = 16
DW = 16
BLK = 1000


def _mesh():
    return plsc.VectorSubcoreMesh(
        core_axis_name="c", subcore_axis_name="s",
        num_cores=NC, num_subcores=NS)


def _deg_body(dst_hbm, out_hbm, didx, ones, zerod, accd, sem):
    c = lax.axis_index("c")
    s = lax.axis_index("s")
    w = c * NS + s
    onev = jnp.ones((16,), jnp.float32)
    zv = jnp.zeros((16,), jnp.float32)

    def fill_ones(i, carry):
        ones[i, :] = onev
        return carry
    lax.fori_loop(0, C, fill_ones, 0)

    def fill_zero(i, carry):
        zerod[i, :] = zv
        return carry
    lax.fori_loop(0, ZR, fill_zero, 0)
    n_init = RB // ZR + jnp.where(s == NS - 1, (N - NS * RB) // ZR, 0)

    def init(j, carry):
        pltpu.async_copy(zerod, accd.at[pl.ds(s * RB + j * ZR, ZR)], sem)
        return carry
    lax.fori_loop(0, n_init, init, 0)

    def init_drain(j, carry):
        pltpu.make_async_copy(zerod, accd.at[pl.ds(s * RB, ZR)], sem).wait()
        return carry
    lax.fori_loop(0, n_init, init_drain, 0)
    plsc.subcore_barrier()

    def sup(u, carry):
        pltpu.sync_copy(dst_hbm.at[w * NSUP + u], didx)

        def chunk(j, carry2):
            pltpu.async_copy(ones, accd.at[didx.at[j]], sem, add=True)
            return carry2
        lax.fori_loop(0, SC, chunk, carry)

        def drain(j, carry2):
            pltpu.make_async_copy(ones, accd.at[didx.at[0]], sem).wait()
            return carry2
        return lax.fori_loop(0, SC, drain, carry)
    lax.fori_loop(0, NSUP, sup, 0)
    plsc.subcore_barrier()
    pltpu.sync_copy(accd.at[pl.ds(s * RB, RB)],
                    out_hbm.at[c, pl.ds(s * RB, RB)])

    @pl.when(s == NS - 1)
    def _():
        pltpu.sync_copy(accd.at[pl.ds(NS * RB, N - NS * RB)],
                        out_hbm.at[c, pl.ds(NS * RB, N - NS * RB)])


def _agg_body(z_hbm, src_hbm, dst_hbm, out_hbm,
              sidx, didx, rows0, rows1, rows2, rows3, rows4, zerov, accsh,
              isem, gsem0, gsem1, gsem2, gsem3, gsem4,
              ssem0, ssem1, ssem2, ssem3, ssem4):
    c = lax.axis_index("c")
    s = lax.axis_index("s")
    w = c * NS + s
    zv = jnp.zeros((16,), jnp.float32)

    def fill_zero(i, carry):
        for j in range(F // 16):
            zerov[i, pl.ds(j * 16, 16)] = zv
        return carry
    lax.fori_loop(0, ZR, fill_zero, 0)
    n_init = RB // ZR + jnp.where(s == NS - 1, (N - NS * RB) // ZR, 0)

    def init(j, carry):
        pltpu.async_copy(zerov, accsh.at[pl.ds(s * RB + j * ZR, ZR)], isem)
        return carry
    lax.fori_loop(0, n_init, init, 0)

    def init_drain(j, carry):
        pltpu.make_async_copy(zerov, accsh.at[pl.ds(s * RB, ZR)], isem).wait()
        return carry
    lax.fori_loop(0, n_init, init_drain, 0)
    plsc.subcore_barrier()

    def refill(u):
        pltpu.sync_copy(src_hbm.at[w * NSUP + u], sidx)
        pltpu.sync_copy(dst_hbm.at[w * NSUP + u], didx)

    def issue_gather(kk):
        jn = kk % SC
        for p in (0, 1):
            @pl.when(kk % 2 == p)
            def _(p=p):
                pltpu.async_copy(z_hbm.at[sidx.at[jn]], rows[p], gsem[p])

    def wait_g(buf, sem):
        pltpu.make_async_copy(z_hbm.at[sidx.at[0]], buf, sem).wait()

    def issue_g(j, buf, sem):
        pltpu.async_copy(z_hbm.at[sidx.at[j]], buf, sem)

    def issue_s(j, buf, sem):
        pltpu.async_copy(buf, accsh.at[didx.at[j]], sem, add=True)

    def wait_s(buf, sem):
        pltpu.make_async_copy(buf, accsh.at[didx.at[0]], sem).wait()

    bufs = (rows0, rows1, rows2, rows3, rows4)
    gsems = (gsem0, gsem1, gsem2, gsem3, gsem4)
    ssems = (ssem0, ssem1, ssem2, ssem3, ssem4)

    def step(j, bj, b3):
        wait_g(bufs[bj], gsems[bj])
        wait_s(bufs[b3], ssems[b3])
        issue_g(j + 3, bufs[b3], gsems[b3])
        issue_s(j, bufs[bj], ssems[bj])

    def sup(u, carry):
        refill(u)
        for b in range(3):
            issue_g(b, bufs[b], gsems[b])
        for j in (0, 1):
            wait_g(bufs[j], gsems[j])
            issue_g(j + 3, bufs[j + 3], gsems[j + 3])
            issue_s(j, bufs[j], ssems[j])

        def penta(t, carry2):
            j0 = 2 + 5 * t
            for b in range(5):
                step(j0 + b, (2 + b) % 5, b)
            return carry2
        lax.fori_loop(0, (SC - 5) // 5, penta, carry)
        for j in range(SC - 3, SC):
            bj = j % 5
            wait_g(bufs[bj], gsems[bj])
            wait_s(bufs[(j + 3) % 5], ssems[(j + 3) % 5])
            issue_s(j, bufs[bj], ssems[bj])
        for j in (SC - 2, SC - 1):
            wait_s(bufs[j % 5], ssems[j % 5])
        return carry
    lax.fori_loop(0, NSUP, sup, 0)
    plsc.subcore_barrier()
    pltpu.sync_copy(accsh.at[pl.ds(s * RB, RB)],
                    out_hbm.at[c, pl.ds(s * RB, RB)])

    @pl.when(s == NS - 1)
    def _():
        pltpu.sync_copy(accsh.at[pl.ds(NS * RB, N - NS * RB)],
                        out_hbm.at[c, pl.ds(NS * RB, N - NS * RB)])


def _deg_call(dst):
    return pl.kernel(
        _deg_body,
        out_type=jax.ShapeDtypeStruct((NC, N, DW), jnp.float32),
        mesh=_mesh(),
        scratch_types=[
            pltpu.VMEM((SC, C), jnp.int32),
            pltpu.VMEM((C, DW), jnp.float32),
            pltpu.VMEM((ZR, DW), jnp.float32),
            pltpu.VMEM_SHARED((N, DW), jnp.float32),
            pltpu.SemaphoreType.DMA,
        ],
    )(dst)


def _agg_call(z, src, dst):
    return pl.kernel(
        _agg_body,
        out_type=jax.ShapeDtypeStruct((NC, N, F), jnp.float32),
        mesh=_mesh(),
        scratch_types=[
            pltpu.VMEM((SC, C), jnp.int32),
            pltpu.VMEM((SC, C), jnp.int32),
            pltpu.VMEM((C, F), jnp.float32),
            pltpu.VMEM((C, F), jnp.float32),
            pltpu.VMEM((C, F), jnp.float32),
            pltpu.VMEM((C, F), jnp.float32),
            pltpu.VMEM((C, F), jnp.float32),
            pltpu.VMEM((ZR, F), jnp.float32),
            pltpu.VMEM_SHARED((N, F), jnp.float32),
        ] + [pltpu.SemaphoreType.DMA] * 11,
    )(z, src, dst)


def _dis(p0_ref, p1_ref):
    deg = 1.0 + p0_ref[0][:, 0:1] + p1_ref[0][:, 0:1]
    return lax.rsqrt(deg)


def _mm1_body(x_ref, w_ref, p0_ref, p1_ref, z_ref):
    dis = _dis(p0_ref, p1_ref)
    y = jnp.dot(x_ref[:], w_ref[:], preferred_element_type=jnp.float32)
    z_ref[:] = y * dis


def _mid_body(a0_ref, a1_ref, z1_ref, p0_ref, p1_ref, b_ref, w_ref, z2_ref):
    dis = _dis(p0_ref, p1_ref)
    h = (a0_ref[0] + a1_ref[0] + z1_ref[:]) * dis + b_ref[:]
    h = jnp.maximum(h, 0.0)
    y = jnp.dot(h, w_ref[:], preferred_element_type=jnp.float32)
    z2_ref[:] = y * dis


def _fin_body(a0_ref, a1_ref, z2_ref, p0_ref, p1_ref, b_ref, mu_ref, ls_ref):
    dis = _dis(p0_ref, p1_ref)
    o = (a0_ref[0] + a1_ref[0] + z2_ref[:]) * dis + b_ref[:]
    mu_ref[:] = o[:, :F // 2]
    ls_ref[:] = o[:, F // 2:]


_row_spec = pl.BlockSpec((BLK, F), lambda i: (i, 0))
_par0_spec = pl.BlockSpec((1, BLK, F), lambda i: (0, i, 0))
_par1_spec = pl.BlockSpec((1, BLK, F), lambda i: (1, i, 0))
_deg0_spec = pl.BlockSpec((1, BLK, DW), lambda i: (0, i, 0))
_deg1_spec = pl.BlockSpec((1, BLK, DW), lambda i: (1, i, 0))
_mat_spec = pl.BlockSpec((F, F), lambda i: (0, 0))
_bias_spec = pl.BlockSpec((1, F), lambda i: (0, 0))
_half_spec = pl.BlockSpec((BLK, F // 2), lambda i: (i, 0))
_out_sds = jax.ShapeDtypeStruct((N, F), jnp.float32)


def _mm1_call(x, W1, degp):
    return pl.pallas_call(
        _mm1_body, grid=(N // BLK,),
        in_specs=[_row_spec, _mat_spec, _deg0_spec, _deg1_spec],
        out_specs=_row_spec, out_shape=_out_sds,
    )(x, W1, degp, degp)


def _mid_call(a, z1, degp, b, Wc):
    return pl.pallas_call(
        _mid_body, grid=(N // BLK,),
        in_specs=[_par0_spec, _par1_spec, _row_spec, _deg0_spec, _deg1_spec,
                  _bias_spec, _mat_spec],
        out_specs=_row_spec, out_shape=_out_sds,
    )(a, a, z1, degp, degp, b, Wc)


def _fin_call(a, z2, degp, b):
    return pl.pallas_call(
        _fin_body, grid=(N // BLK,),
        in_specs=[_par0_spec, _par1_spec, _row_spec, _deg0_spec, _deg1_spec,
                  _bias_spec],
        out_specs=[_half_spec, _half_spec],
        out_shape=(jax.ShapeDtypeStruct((N, F // 2), jnp.float32),
                   jax.ShapeDtypeStruct((N, F // 2), jnp.float32)),
    )(a, a, z2, degp, degp, b)


def kernel(x, edge_index, W1, b1, W_mu, b_mu, W_ls, b_ls):
    src = edge_index[0].astype(jnp.int32).reshape(NW * NSUP, SC, C)
    dst = edge_index[1].astype(jnp.int32).reshape(NW * NSUP, SC, C)

    degp = _deg_call(dst)

    Wc = jnp.concatenate([W_mu, W_ls], axis=1)
    b1r = b1.reshape(1, F)
    bcr = jnp.concatenate([b_mu, b_ls]).reshape(1, F)

    z1 = _mm1_call(x, W1, degp)
    a1 = _agg_call(z1, src, dst)
    z2 = _mid_call(a1, z1, degp, b1r, Wc)
    a2 = _agg_call(z2, src, dst)
    mu, ls = _fin_call(a2, z2, degp, bcr)
    return mu, ls


# speedup vs baseline: 1.1167x; 1.1167x over previous
import jax
import jax.numpy as jnp
from jax import lax
from jax.experimental import pallas as pl
from jax.experimental.pallas import tpu as pltpu
from jax.experimental.pallas import tpu_sc as plsc

N = 10000
F = 128
E = 320000
NC = 2
NS = 16
NW = NC * NS
PW = E // NW
C = 50
NCHUNK = PW // C
RB = 624
SC = 40
NSUP = NCHUNK // SC
ZR = 16
DW = 16
BLK = 1000


def _mesh():
    return plsc.VectorSubcoreMesh(
        core_axis_name="c", subcore_axis_name="s",
        num_cores=NC, num_subcores=NS)


def _deg_body(dst_hbm, out_hbm, didx, ones, zerod, accd, sem):
    c = lax.axis_index("c")
    s = lax.axis_index("s")
    w = c * NS + s
    onev = jnp.ones((16,), jnp.float32)
    zv = jnp.zeros((16,), jnp.float32)

    def fill_ones(i, carry):
        ones[i, :] = onev
        return carry
    lax.fori_loop(0, C, fill_ones, 0)

    def fill_zero(i, carry):
        zerod[i, :] = zv
        return carry
    lax.fori_loop(0, ZR, fill_zero, 0)
    n_init = RB // ZR + jnp.where(s == NS - 1, (N - NS * RB) // ZR, 0)

    def init(j, carry):
        pltpu.async_copy(zerod, accd.at[pl.ds(s * RB + j * ZR, ZR)], sem)
        return carry
    lax.fori_loop(0, n_init, init, 0)

    def init_drain(j, carry):
        pltpu.make_async_copy(zerod, accd.at[pl.ds(s * RB, ZR)], sem).wait()
        return carry
    lax.fori_loop(0, n_init, init_drain, 0)
    plsc.subcore_barrier()

    def sup(u, carry):
        pltpu.sync_copy(dst_hbm.at[w * NSUP + u], didx)

        def chunk(j, carry2):
            pltpu.async_copy(ones, accd.at[didx.at[j]], sem, add=True)
            return carry2
        lax.fori_loop(0, SC, chunk, carry)

        def drain(j, carry2):
            pltpu.make_async_copy(ones, accd.at[didx.at[0]], sem).wait()
            return carry2
        return lax.fori_loop(0, SC, drain, carry)
    lax.fori_loop(0, NSUP, sup, 0)
    plsc.subcore_barrier()
    pltpu.sync_copy(accd.at[pl.ds(s * RB, RB)],
                    out_hbm.at[c, pl.ds(s * RB, RB)])

    @pl.when(s == NS - 1)
    def _():
        pltpu.sync_copy(accd.at[pl.ds(NS * RB, N - NS * RB)],
                        out_hbm.at[c, pl.ds(NS * RB, N - NS * RB)])


def _agg_body(z_hbm, src_hbm, dst_hbm, out_hbm,
              sidx, didx, rows0, rows1, rows2, rows3, zerov, accsh,
              isem, gsem0, gsem1, gsem2, gsem3, ssem0):
    c = lax.axis_index("c")
    s = lax.axis_index("s")
    w = c * NS + s
    zv = jnp.zeros((16,), jnp.float32)

    def fill_zero(i, carry):
        for j in range(F // 16):
            zerov[i, pl.ds(j * 16, 16)] = zv
        return carry
    lax.fori_loop(0, ZR, fill_zero, 0)
    n_init = RB // ZR + jnp.where(s == NS - 1, (N - NS * RB) // ZR, 0)

    def init(j, carry):
        pltpu.async_copy(zerov, accsh.at[pl.ds(s * RB + j * ZR, ZR)], isem)
        return carry
    lax.fori_loop(0, n_init, init, 0)

    def init_drain(j, carry):
        pltpu.make_async_copy(zerov, accsh.at[pl.ds(s * RB, ZR)], isem).wait()
        return carry
    lax.fori_loop(0, n_init, init_drain, 0)
    plsc.subcore_barrier()

    def refill(u):
        pltpu.sync_copy(src_hbm.at[w * NSUP + u], sidx)
        pltpu.sync_copy(dst_hbm.at[w * NSUP + u], didx)

    def issue_gather(kk):
        jn = kk % SC
        for p in (0, 1):
            @pl.when(kk % 2 == p)
            def _(p=p):
                pltpu.async_copy(z_hbm.at[sidx.at[jn]], rows[p], gsem[p])

    def wait_g(buf, sem):
        pltpu.make_async_copy(z_hbm.at[sidx.at[0]], buf, sem).wait()

    def issue_g(j, buf, sem):
        pltpu.async_copy(z_hbm.at[sidx.at[j]], buf, sem)

    def scatter(j, buf):
        pltpu.async_copy(buf, accsh.at[didx.at[j]], ssem0, add=True).wait()

    bufs = (rows0, rows1, rows2, rows3)
    sems = (gsem0, gsem1, gsem2, gsem3)

    def sup(u, carry):
        refill(u)
        for b in range(3):
            issue_g(b, bufs[b], sems[b])

        def quad(q, carry2):
            j = 4 * q
            for b in range(4):
                wait_g(bufs[b], sems[b])
                issue_g(j + b + 3, bufs[(b + 3) % 4], sems[(b + 3) % 4])
                scatter(j + b, bufs[b])
            return carry2
        lax.fori_loop(0, (SC - 4) // 4, quad, carry)
        j = SC - 4
        wait_g(bufs[0], sems[0])
        issue_g(SC - 1, bufs[3], sems[3])
        scatter(j, bufs[0])
        for b in range(1, 4):
            wait_g(bufs[b], sems[b])
            scatter(j + b, bufs[b])
        return carry
    lax.fori_loop(0, NSUP, sup, 0)
    plsc.subcore_barrier()
    pltpu.sync_copy(accsh.at[pl.ds(s * RB, RB)],
                    out_hbm.at[c, pl.ds(s * RB, RB)])

    @pl.when(s == NS - 1)
    def _():
        pltpu.sync_copy(accsh.at[pl.ds(NS * RB, N - NS * RB)],
                        out_hbm.at[c, pl.ds(NS * RB, N - NS * RB)])


def _deg_call(dst):
    return pl.kernel(
        _deg_body,
        out_type=jax.ShapeDtypeStruct((NC, N, DW), jnp.float32),
        mesh=_mesh(),
        scratch_types=[
            pltpu.VMEM((SC, C), jnp.int32),
            pltpu.VMEM((C, DW), jnp.float32),
            pltpu.VMEM((ZR, DW), jnp.float32),
            pltpu.VMEM_SHARED((N, DW), jnp.float32),
            pltpu.SemaphoreType.DMA,
        ],
    )(dst)


def _agg_call(z, src, dst):
    return pl.kernel(
        _agg_body,
        out_type=jax.ShapeDtypeStruct((NC, N, F), jnp.float32),
        mesh=_mesh(),
        scratch_types=[
            pltpu.VMEM((SC, C), jnp.int32),
            pltpu.VMEM((SC, C), jnp.int32),
            pltpu.VMEM((C, F), jnp.float32),
            pltpu.VMEM((C, F), jnp.float32),
            pltpu.VMEM((C, F), jnp.float32),
            pltpu.VMEM((C, F), jnp.float32),
            pltpu.VMEM((ZR, F), jnp.float32),
            pltpu.VMEM_SHARED((N, F), jnp.float32),
        ] + [pltpu.SemaphoreType.DMA] * 6,
    )(z, src, dst)


def _dis(p0_ref, p1_ref):
    deg = 1.0 + p0_ref[0][:, 0:1] + p1_ref[0][:, 0:1]
    return lax.rsqrt(deg)


def _mm1_body(x_ref, w_ref, y_ref):
    y_ref[:] = jnp.dot(x_ref[:], w_ref[:],
                       preferred_element_type=jnp.float32)


def _scale_body(y_ref, p0_ref, p1_ref, z_ref):
    z_ref[:] = y_ref[:] * _dis(p0_ref, p1_ref)


def _mid_body(a0_ref, a1_ref, z1_ref, p0_ref, p1_ref, b_ref, w_ref, z2_ref):
    dis = _dis(p0_ref, p1_ref)
    h = (a0_ref[0] + a1_ref[0] + z1_ref[:]) * dis + b_ref[:]
    h = jnp.maximum(h, 0.0)
    y = jnp.dot(h, w_ref[:], preferred_element_type=jnp.float32)
    z2_ref[:] = y * dis


def _fin_body(a0_ref, a1_ref, z2_ref, p0_ref, p1_ref, b_ref, mu_ref, ls_ref):
    dis = _dis(p0_ref, p1_ref)
    o = (a0_ref[0] + a1_ref[0] + z2_ref[:]) * dis + b_ref[:]
    mu_ref[:] = o[:, :F // 2]
    ls_ref[:] = o[:, F // 2:]


_row_spec = pl.BlockSpec((BLK, F), lambda i: (i, 0))
_par0_spec = pl.BlockSpec((1, BLK, F), lambda i: (0, i, 0))
_par1_spec = pl.BlockSpec((1, BLK, F), lambda i: (1, i, 0))
_deg0_spec = pl.BlockSpec((1, BLK, DW), lambda i: (0, i, 0))
_deg1_spec = pl.BlockSpec((1, BLK, DW), lambda i: (1, i, 0))
_mat_spec = pl.BlockSpec((F, F), lambda i: (0, 0))
_bias_spec = pl.BlockSpec((1, F), lambda i: (0, 0))
_half_spec = pl.BlockSpec((BLK, F // 2), lambda i: (i, 0))
_out_sds = jax.ShapeDtypeStruct((N, F), jnp.float32)


def _mm1_call(x, W1):
    return pl.pallas_call(
        _mm1_body, grid=(N // BLK,),
        in_specs=[_row_spec, _mat_spec],
        out_specs=_row_spec, out_shape=_out_sds,
    )(x, W1)


def _scale_call(y, degp):
    return pl.pallas_call(
        _scale_body, grid=(N // BLK,),
        in_specs=[_row_spec, _deg0_spec, _deg1_spec],
        out_specs=_row_spec, out_shape=_out_sds,
    )(y, degp, degp)


def _mid_call(a, z1, degp, b, Wc):
    return pl.pallas_call(
        _mid_body, grid=(N // BLK,),
        in_specs=[_par0_spec, _par1_spec, _row_spec, _deg0_spec, _deg1_spec,
                  _bias_spec, _mat_spec],
        out_specs=_row_spec, out_shape=_out_sds,
    )(a, a, z1, degp, degp, b, Wc)


def _fin_call(a, z2, degp, b):
    return pl.pallas_call(
        _fin_body, grid=(N // BLK,),
        in_specs=[_par0_spec, _par1_spec, _row_spec, _deg0_spec, _deg1_spec,
                  _bias_spec],
        out_specs=[_half_spec, _half_spec],
        out_shape=(jax.ShapeDtypeStruct((N, F // 2), jnp.float32),
                   jax.ShapeDtypeStruct((N, F // 2), jnp.float32)),
    )(a, a, z2, degp, degp, b)


def kernel(x, edge_index, W1, b1, W_mu, b_mu, W_ls, b_ls):
    src = edge_index[0].astype(jnp.int32).reshape(NW * NSUP, SC, C)
    dst = edge_index[1].astype(jnp.int32).reshape(NW * NSUP, SC, C)

    degp = _deg_call(dst)

    Wc = jnp.concatenate([W_mu, W_ls], axis=1)
    b1r = b1.reshape(1, F)
    bcr = jnp.concatenate([b_mu, b_ls]).reshape(1, F)

    y1 = _mm1_call(x, W1)
    z1 = _scale_call(y1, degp)
    a1 = _agg_call(z1, src, dst)
    z2 = _mid_call(a1, z1, degp, b1r, Wc)
    a2 = _agg_call(z2, src, dst)
    mu, ls = _fin_call(a2, z2, degp, bcr)
    return mu, ls
